# baseline (speedup 1.0000x reference)
import functools

import jax
import jax.numpy as jnp
from jax import lax
from jax.experimental import pallas as pl
from jax.experimental.pallas import tpu as pltpu
from jax.experimental.pallas import tpu_sc as plsc

ROWS = 16384
NW = 16
NBLK = ROWS // 128
BLK_PER = NBLK // NW
IN_PER = BLK_PER * 256
OUT_PER = BLK_PER * 512

_WA_OFF = 1
_WB_OFF = 4
_B_OFF = 7
_EMB_OFF = 10


def _sc_body(x_hbm, p_hbm, out_hbm, xv, ov, pv, s0, s1):
    wid = lax.axis_index("s")

    cp_x = pltpu.async_copy(x_hbm.at[pl.ds(wid * IN_PER, IN_PER)], xv, s0)
    pltpu.sync_copy(p_hbm, pv.at[pl.ds(0, 48)])

    zero = lax.iota(jnp.int32, 16) * 0
    wa, wb, cc = [], [], []
    for c in range(3):
        wa.append(plsc.load_gather(pv, [zero + (_WA_OFF + c)]))
        wb.append(plsc.load_gather(pv, [zero + (_WB_OFF + c)]))
        bc = plsc.load_gather(pv, [zero + (_B_OFF + c)])
        e0 = plsc.load_gather(pv, [zero + (_EMB_OFF + 10 * c)])
        cc.append(bc + e0)

    cp_x.wait()

    half = BLK_PER * 4

    @plsc.parallel_loop(0, half, unroll=2)
    def body0(i):
        kk = lax.shift_right_logical(i, 3)
        v = lax.bitwise_and(i, 7)
        in0 = 256 * kk + 16 * v
        o0 = 512 * kk + 16 * v
        av = xv[pl.ds(in0, 16)]
        bv = xv[pl.ds(in0 + 128, 16)]
        for c in range(3):
            ov[pl.ds(o0 + 128 * c, 16)] = av * wa[c] + bv * wb[c] + cc[c]

    cp_o0 = pltpu.async_copy(
        ov.at[pl.ds(0, OUT_PER // 2)],
        out_hbm.at[pl.ds(wid * OUT_PER, OUT_PER // 2)],
        s1,
    )

    @plsc.parallel_loop(half, BLK_PER * 8, unroll=2)
    def body1(i):
        kk = lax.shift_right_logical(i, 3)
        v = lax.bitwise_and(i, 7)
        in0 = 256 * kk + 16 * v
        o0 = 512 * kk + 16 * v
        av = xv[pl.ds(in0, 16)]
        bv = xv[pl.ds(in0 + 128, 16)]
        for c in range(3):
            ov[pl.ds(o0 + 128 * c, 16)] = av * wa[c] + bv * wb[c] + cc[c]

    pltpu.sync_copy(
        ov.at[pl.ds(OUT_PER // 2, OUT_PER // 2)],
        out_hbm.at[pl.ds(wid * OUT_PER + OUT_PER // 2, OUT_PER // 2)],
    )
    cp_o0.wait()


def _tc_pack(wt_ref, b_ref, embt_ref, o_ref):
    o_ref[pl.ds(0, 1)] = jnp.zeros((1,), jnp.float32)
    o_ref[pl.ds(_WA_OFF, 3)] = wt_ref[0]
    o_ref[pl.ds(_WB_OFF, 3)] = wt_ref[1]
    o_ref[pl.ds(_B_OFF, 3)] = b_ref[...]
    for c in range(3):
        o_ref[pl.ds(_EMB_OFF + 10 * c, 10)] = embt_ref[c]
    o_ref[pl.ds(40, 8)] = jnp.zeros((8,), jnp.float32)


@jax.jit
def kernel(x, W, b, emb):
    x1d = x.reshape(NBLK, 128, 2).transpose(0, 2, 1).reshape(-1)
    params = pl.pallas_call(
        _tc_pack,
        out_shape=jax.ShapeDtypeStruct((48,), jnp.float32),
    )(W.T, b, emb.T)
    mesh = plsc.VectorSubcoreMesh(
        core_axis_name="c", subcore_axis_name="s", num_cores=1
    )
    run = functools.partial(
        pl.kernel,
        mesh=mesh,
        out_type=jax.ShapeDtypeStruct((ROWS * 4,), jnp.float32),
        compiler_params=pltpu.CompilerParams(needs_layout_passes=False),
        scratch_types=[
            pltpu.VMEM((IN_PER,), jnp.float32),
            pltpu.VMEM((OUT_PER,), jnp.float32),
            pltpu.VMEM((128,), jnp.float32),
            pltpu.SemaphoreType.DMA,
            pltpu.SemaphoreType.DMA,
        ],
    )(_sc_body)
    out1d = run(x1d, params)
    return out1d.reshape(NBLK, 4, 128)[:, :3, :].transpose(0, 2, 1).reshape(ROWS, 3)


# speedup vs baseline: 1.0223x; 1.0223x over previous
import functools

import jax
import jax.numpy as jnp
from jax import lax
from jax.experimental import pallas as pl
from jax.experimental.pallas import tpu as pltpu
from jax.experimental.pallas import tpu_sc as plsc

ROWS = 16384
NW = 16
NBLK = ROWS // 128
BLK_PER = NBLK // NW
IN_PER = BLK_PER * 256
OUT_PER = BLK_PER * 512

_WA_OFF = 1
_WB_OFF = 4
_B_OFF = 7
_EMB_OFF = 10


def _sc_body(x_hbm, p_hbm, out_hbm, xv, ov, pv, s0):
    wid = lax.axis_index("s")

    cp_x = pltpu.async_copy(x_hbm.at[pl.ds(wid * IN_PER, IN_PER)], xv, s0)
    pltpu.sync_copy(p_hbm, pv.at[pl.ds(0, 48)])

    zero = lax.iota(jnp.int32, 16) * 0
    wa, wb, cc = [], [], []
    for c in range(3):
        wa.append(plsc.load_gather(pv, [zero + (_WA_OFF + c)]))
        wb.append(plsc.load_gather(pv, [zero + (_WB_OFF + c)]))
        bc = plsc.load_gather(pv, [zero + (_B_OFF + c)])
        e0 = plsc.load_gather(pv, [zero + (_EMB_OFF + 10 * c)])
        cc.append(bc + e0)

    cp_x.wait()

    @plsc.parallel_loop(0, BLK_PER * 8, unroll=2)
    def body(i):
        kk = lax.shift_right_logical(i, 3)
        v = lax.bitwise_and(i, 7)
        in0 = 256 * kk + 16 * v
        o0 = 512 * kk + 16 * v
        av = xv[pl.ds(in0, 16)]
        bv = xv[pl.ds(in0 + 128, 16)]
        for c in range(3):
            ov[pl.ds(o0 + 128 * c, 16)] = av * wa[c] + bv * wb[c] + cc[c]

    pltpu.sync_copy(ov, out_hbm.at[pl.ds(wid * OUT_PER, OUT_PER)])


def _tc_pack(wt_ref, b_ref, embt_ref, o_ref):
    o_ref[pl.ds(0, 1)] = jnp.zeros((1,), jnp.float32)
    o_ref[pl.ds(_WA_OFF, 3)] = wt_ref[0]
    o_ref[pl.ds(_WB_OFF, 3)] = wt_ref[1]
    o_ref[pl.ds(_B_OFF, 3)] = b_ref[...]
    for c in range(3):
        o_ref[pl.ds(_EMB_OFF + 10 * c, 10)] = embt_ref[c]
    o_ref[pl.ds(40, 8)] = jnp.zeros((8,), jnp.float32)


@jax.jit
def kernel(x, W, b, emb):
    x1d = x.reshape(NBLK, 128, 2).transpose(0, 2, 1).reshape(-1)
    params = pl.pallas_call(
        _tc_pack,
        out_shape=jax.ShapeDtypeStruct((48,), jnp.float32),
    )(W.T, b, emb.T)
    mesh = plsc.VectorSubcoreMesh(
        core_axis_name="c", subcore_axis_name="s", num_cores=1
    )
    run = functools.partial(
        pl.kernel,
        mesh=mesh,
        out_type=jax.ShapeDtypeStruct((ROWS * 4,), jnp.float32),
        compiler_params=pltpu.CompilerParams(needs_layout_passes=False),
        scratch_types=[
            pltpu.VMEM((IN_PER,), jnp.float32),
            pltpu.VMEM((OUT_PER,), jnp.float32),
            pltpu.VMEM((128,), jnp.float32),
            pltpu.SemaphoreType.DMA,
        ],
    )(_sc_body)
    out1d = run(x1d, params)
    return out1d.reshape(NBLK, 4, 128)[:, :3, :].transpose(0, 2, 1).reshape(ROWS, 3)
